# Initial kernel scaffold; baseline (speedup 1.0000x reference)
import jax
import jax.numpy as jnp
from jax.experimental import pallas as pl


def kernel(input, embed):
    raise NotImplementedError("write your pallas kernel here")


# speedup vs baseline: 1.3160x; 1.3160x over previous
import functools

import jax
import jax.numpy as jnp
from jax import lax
from jax.experimental import pallas as pl
from jax.experimental.pallas import tpu as pltpu
from jax.experimental.pallas import tpu_sc as plsc

DIM = 64
N_EMBED = 1024


_BLK = 2048


def _dist_argmin_kernel(x_ref, e_ref, ind_ref, diff_ref):
    i = pl.program_id(0)
    nsteps = pl.num_programs(0)
    x = x_ref[...]
    e = e_ref[...]
    xe = jnp.dot(x, e, preferred_element_type=jnp.float32)
    x2 = jnp.sum(x * x, axis=1, keepdims=True)
    e2 = jnp.sum(e * e, axis=0, keepdims=True)
    d2 = x2 - 2.0 * xe + e2
    idx = jnp.argmin(d2, axis=1)
    minv = jnp.min(d2, axis=1)
    ind_ref[0, 0, :] = idx
    part = jnp.sum(jnp.maximum(minv, 0.0), keepdims=True).reshape(1, 1)

    @pl.when(i == 0)
    def _():
        diff_ref[...] = jnp.zeros((1, 1), jnp.float32)

    diff_ref[...] += part

    @pl.when(i == nsteps - 1)
    def _():
        diff_ref[...] = diff_ref[...] / float(_TOTAL_ELEMS)


_TOTAL_ELEMS = 64 * 1024 * DIM


def _dist_argmin(x, embed):
    n = x.shape[0]
    nblk = n // _BLK
    return pl.pallas_call(
        _dist_argmin_kernel,
        grid=(nblk,),
        in_specs=[
            pl.BlockSpec((_BLK, DIM), lambda i: (i, 0)),
            pl.BlockSpec((DIM, N_EMBED), lambda i: (0, 0)),
        ],
        out_specs=[
            pl.BlockSpec((1, 1, _BLK), lambda i: (i, 0, 0)),
            pl.BlockSpec((1, 1), lambda i: (0, 0)),
        ],
        out_shape=[
            jax.ShapeDtypeStruct((nblk, 1, _BLK), jnp.int32),
            jax.ShapeDtypeStruct((1, 1), jnp.float32),
        ],
    )(x, embed)


_SC_CHUNK = 512


def _make_sc_gather(n):
    info = plsc.get_sparse_core_info()
    nc, ns = info.num_cores, info.num_subcores
    nw = nc * ns
    b_per_w = n // nw
    nchunk = b_per_w // _SC_CHUNK
    mesh = plsc.VectorSubcoreMesh(core_axis_name="c", subcore_axis_name="s")

    @functools.partial(
        pl.kernel,
        mesh=mesh,
        compiler_params=pltpu.CompilerParams(use_tc_tiling_on_sc=False),
        out_type=jax.ShapeDtypeStruct((n, DIM), jnp.float32),
        scratch_types=[
            pltpu.VMEM((_SC_CHUNK,), jnp.int32),
            pltpu.VMEM((_SC_CHUNK, DIM), jnp.float32),
            pltpu.SemaphoreType.DMA,
        ],
    )
    def sc_gather(table_hbm, idx_hbm, out_hbm, idx_v, rows_v, sem):
        wid = lax.axis_index("s") * nc + lax.axis_index("c")
        base = wid * b_per_w
        for c in range(nchunk):
            off = base + c * _SC_CHUNK
            pltpu.sync_copy(idx_hbm.at[pl.ds(off, _SC_CHUNK)], idx_v)
            pltpu.async_copy(table_hbm.at[idx_v], rows_v, sem).wait()
            pltpu.sync_copy(rows_v, out_hbm.at[pl.ds(off, _SC_CHUNK)])

    return sc_gather


def kernel(input, embed):
    dim = embed.shape[0]
    x = input.reshape(-1, dim)
    n = x.shape[0]
    ind3, diff11 = _dist_argmin(x, embed)
    ind_flat = ind3.reshape(-1)
    table = embed.T
    q = _make_sc_gather(n)(table, ind_flat)
    quantize = q.reshape(input.shape)
    diff = diff11.reshape(())
    embed_ind = ind_flat.reshape(input.shape[:-1])
    return (quantize, diff, embed_ind)


# speedup vs baseline: 1.3420x; 1.0197x over previous
import functools

import jax
import jax.numpy as jnp
from jax import lax
from jax.experimental import pallas as pl
from jax.experimental.pallas import tpu as pltpu
from jax.experimental.pallas import tpu_sc as plsc

DIM = 64
N_EMBED = 1024


_BLK = 2048


def _dist_argmin_kernel(x_ref, e_ref, ind_ref, diff_ref):
    i = pl.program_id(0)
    nsteps = pl.num_programs(0)
    x = x_ref[...]
    e = e_ref[...]
    em2 = e * (-2.0)
    xe2 = jnp.dot(x, em2, preferred_element_type=jnp.float32)
    e2 = jnp.sum(e * e, axis=0, keepdims=True)
    s = xe2 + e2
    idx = jnp.argmin(s, axis=1)
    x2v = jnp.sum(x * x, axis=1)
    minv = jnp.min(s, axis=1) + x2v
    ind_ref[0, 0, :] = idx
    part = jnp.sum(jnp.maximum(minv, 0.0), keepdims=True).reshape(1, 1)

    @pl.when(i == 0)
    def _():
        diff_ref[...] = jnp.zeros((1, 1), jnp.float32)

    diff_ref[...] += part

    @pl.when(i == nsteps - 1)
    def _():
        diff_ref[...] = diff_ref[...] / float(_TOTAL_ELEMS)


_TOTAL_ELEMS = 64 * 1024 * DIM


def _dist_argmin(x, embed):
    n = x.shape[0]
    nblk = n // _BLK
    return pl.pallas_call(
        _dist_argmin_kernel,
        grid=(nblk,),
        in_specs=[
            pl.BlockSpec((_BLK, DIM), lambda i: (i, 0)),
            pl.BlockSpec((DIM, N_EMBED), lambda i: (0, 0)),
        ],
        out_specs=[
            pl.BlockSpec((1, 1, _BLK), lambda i: (i, 0, 0)),
            pl.BlockSpec((1, 1), lambda i: (0, 0)),
        ],
        out_shape=[
            jax.ShapeDtypeStruct((nblk, 1, _BLK), jnp.int32),
            jax.ShapeDtypeStruct((1, 1), jnp.float32),
        ],
    )(x, embed)


_SC_CHUNK = 512


def _make_sc_gather(n):
    info = plsc.get_sparse_core_info()
    nc, ns = info.num_cores, info.num_subcores
    nw = nc * ns
    b_per_w = n // nw
    nchunk = b_per_w // _SC_CHUNK
    mesh = plsc.VectorSubcoreMesh(core_axis_name="c", subcore_axis_name="s")

    @functools.partial(
        pl.kernel,
        mesh=mesh,
        compiler_params=pltpu.CompilerParams(use_tc_tiling_on_sc=False),
        out_type=jax.ShapeDtypeStruct((n, DIM), jnp.float32),
        scratch_types=[
            pltpu.VMEM((2, _SC_CHUNK), jnp.int32),
            pltpu.VMEM((2, _SC_CHUNK, DIM), jnp.float32),
            [pltpu.SemaphoreType.DMA] * 2,
            [pltpu.SemaphoreType.DMA] * 2,
            [pltpu.SemaphoreType.DMA] * 2,
        ],
    )
    def sc_gather(table_hbm, idx_hbm, out_hbm, idx_v, rows_v, si, sg, sw):
        wid = lax.axis_index("s") * nc + lax.axis_index("c")
        base = wid * b_per_w

        def osl(c):
            return pl.ds(base + c * _SC_CHUNK, _SC_CHUNK)

        icp = [None] * nchunk
        gcp = [None] * nchunk
        wcp = [None] * nchunk
        for c in range(min(2, nchunk)):
            icp[c] = pltpu.async_copy(idx_hbm.at[osl(c)], idx_v.at[c % 2], si[c % 2])
        icp[0].wait()
        gcp[0] = pltpu.async_copy(table_hbm.at[idx_v.at[0]], rows_v.at[0], sg[0])
        for c in range(nchunk):
            b = c % 2
            gcp[c].wait()
            wcp[c] = pltpu.async_copy(rows_v.at[b], out_hbm.at[osl(c)], sw[b])
            if c + 1 < nchunk:
                if c - 1 >= 0:
                    wcp[c - 1].wait()
                icp[c + 1].wait()
                gcp[c + 1] = pltpu.async_copy(
                    table_hbm.at[idx_v.at[1 - b]], rows_v.at[1 - b], sg[1 - b]
                )
                if c + 2 < nchunk:
                    icp[c + 2] = pltpu.async_copy(
                        idx_hbm.at[osl(c + 2)], idx_v.at[b], si[b]
                    )
        if nchunk >= 2:
            wcp[nchunk - 2].wait()
        wcp[nchunk - 1].wait()

    return sc_gather


def kernel(input, embed):
    dim = embed.shape[0]
    x = input.reshape(-1, dim)
    n = x.shape[0]
    ind3, diff11 = _dist_argmin(x, embed)
    ind_flat = ind3.reshape(-1)
    table = embed.T
    q = _make_sc_gather(n)(table, ind_flat)
    quantize = q.reshape(input.shape)
    diff = diff11.reshape(())
    embed_ind = ind_flat.reshape(input.shape[:-1])
    return (quantize, diff, embed_ind)


# speedup vs baseline: 1.7346x; 1.2926x over previous
import functools

import jax
import jax.numpy as jnp
from jax import lax
from jax.experimental import pallas as pl
from jax.experimental.pallas import tpu as pltpu
from jax.experimental.pallas import tpu_sc as plsc

DIM = 64
N_EMBED = 1024


_BLK = 2048


def _dist_argmin_kernel(x_ref, e_ref, ind_ref, diff_ref):
    i = pl.program_id(0)
    nsteps = pl.num_programs(0)
    x = x_ref[...]
    e = e_ref[...]
    em2 = e * (-2.0)
    xe2 = jnp.dot(x, em2, preferred_element_type=jnp.float32)
    e2 = jnp.sum(e * e, axis=0, keepdims=True)
    s = xe2 + e2
    idx = jnp.argmin(s, axis=1)
    x2v = jnp.sum(x * x, axis=1)
    minv = jnp.min(s, axis=1) + x2v
    ind_ref[0, 0, :] = idx
    part = jnp.sum(jnp.maximum(minv, 0.0), keepdims=True).reshape(1, 1)

    @pl.when(i == 0)
    def _():
        diff_ref[...] = jnp.zeros((1, 1), jnp.float32)

    diff_ref[...] += part

    @pl.when(i == nsteps - 1)
    def _():
        diff_ref[...] = diff_ref[...] / float(_TOTAL_ELEMS)


_TOTAL_ELEMS = 64 * 1024 * DIM


def _dist_argmin(x, embed):
    n = x.shape[0]
    nblk = n // _BLK
    return pl.pallas_call(
        _dist_argmin_kernel,
        grid=(nblk,),
        in_specs=[
            pl.BlockSpec((_BLK, DIM), lambda i: (i, 0)),
            pl.BlockSpec((DIM, N_EMBED), lambda i: (0, 0)),
        ],
        out_specs=[
            pl.BlockSpec((1, 1, _BLK), lambda i: (i, 0, 0)),
            pl.BlockSpec((1, 1), lambda i: (0, 0)),
        ],
        out_shape=[
            jax.ShapeDtypeStruct((nblk, 1, _BLK), jnp.int32),
            jax.ShapeDtypeStruct((1, 1), jnp.float32),
        ],
    )(x, embed)


_SC_CHUNK = 256


def _make_sc_gather(n):
    info = plsc.get_sparse_core_info()
    nc, ns = info.num_cores, info.num_subcores
    nw = nc * ns
    b_per_w = n // nw
    nchunk = b_per_w // _SC_CHUNK
    mesh = plsc.VectorSubcoreMesh(core_axis_name="c", subcore_axis_name="s")

    @functools.partial(
        pl.kernel,
        mesh=mesh,
        compiler_params=pltpu.CompilerParams(use_tc_tiling_on_sc=False),
        out_type=jax.ShapeDtypeStruct((n, DIM), jnp.float32),
        scratch_types=[
            pltpu.VMEM((N_EMBED * DIM,), jnp.float32),
            pltpu.VMEM((b_per_w,), jnp.int32),
            pltpu.VMEM((2, _SC_CHUNK, DIM), jnp.float32),
            pltpu.SemaphoreType.DMA,
            pltpu.SemaphoreType.DMA,
            [pltpu.SemaphoreType.DMA] * 2,
        ],
    )
    def sc_gather(table_hbm, idx_hbm, out_hbm, table_v, idx_v, out_v, st, si, sw):
        wid = lax.axis_index("s") * nc + lax.axis_index("c")
        base = wid * b_per_w
        tcp = pltpu.async_copy(table_hbm, table_v, st)
        icp = pltpu.async_copy(idx_hbm.at[pl.ds(base, b_per_w)], idx_v, si)
        tcp.wait()
        icp.wait()
        wcp = [None] * nchunk
        for c in range(nchunk):
            b = c % 2
            if c >= 2:
                wcp[c - 2].wait()

            @functools.partial(plsc.parallel_loop, 0, _SC_CHUNK, unroll=4)
            def _(t):
                a = idx_v[c * _SC_CHUNK + t] * DIM
                for j in range(0, DIM, 16):
                    out_v[b, t, pl.ds(j, 16)] = table_v[pl.ds(a + j, 16)]

            wcp[c] = pltpu.async_copy(
                out_v.at[b], out_hbm.at[pl.ds(base + c * _SC_CHUNK, _SC_CHUNK)], sw[b]
            )
        wcp[nchunk - 2].wait()
        wcp[nchunk - 1].wait()

    return sc_gather


def kernel(input, embed):
    dim = embed.shape[0]
    x = input.reshape(-1, dim)
    n = x.shape[0]
    ind3, diff11 = _dist_argmin(x, embed)
    ind_flat = ind3.reshape(-1)
    table = embed.T.reshape(-1)
    q = _make_sc_gather(n)(table, ind_flat)
    quantize = q.reshape(input.shape)
    diff = diff11.reshape(())
    embed_ind = ind_flat.reshape(input.shape[:-1])
    return (quantize, diff, embed_ind)
